# baseline (speedup 1.0000x reference)
import functools

import jax
import jax.numpy as jnp
from jax import lax
from jax.experimental import pallas as pl
from jax.experimental.pallas import tpu as pltpu
from jax.experimental.pallas import tpu_sc as plsc

NUM_EMB = 1000000
DIM = 32
ROWS128 = NUM_EMB * DIM // 128
BLK = 2000
N_BLOCKS = ROWS128 // BLK

BATCH = 16384
HIST = 50
NW = 32
B_PER_W = BATCH // NW
NB = 16
N_CHUNKS = B_PER_W // NB


def _maxabs_body(x_ref, o_ref):
    i = pl.program_id(0)

    @pl.when(i == 0)
    def _init():
        o_ref[...] = jnp.zeros((1, 1), jnp.float32)

    o_ref[...] = jnp.maximum(o_ref[...], jnp.max(jnp.abs(x_ref[...])))


def _quant_body(s_ref, x_ref, o_ref):
    scale = jnp.maximum(s_ref[...] / 7.0, 1e-08)
    o_ref[...] = jnp.clip(jnp.round(x_ref[...] / scale), -8.0, 7.0) * scale


@functools.cache
def _make_gather():
    mesh = plsc.VectorSubcoreMesh(core_axis_name="c", subcore_axis_name="s")

    @functools.partial(
        pl.kernel,
        mesh=mesh,
        compiler_params=pltpu.CompilerParams(use_tc_tiling_on_sc=False),
        out_type=jax.ShapeDtypeStruct((BATCH, HIST, DIM), jnp.float32),
        scratch_types=[
            pltpu.VMEM((NB, HIST), jnp.int32),
            pltpu.VMEM((NB, HIST, DIM), jnp.float32),
            pltpu.SemaphoreType.DMA,
        ],
    )
    def gather_k(table_hbm, idx_hbm, out_hbm, idx_v, rows_v, sem):
        wid = lax.axis_index("s") * 2 + lax.axis_index("c")
        base = wid * B_PER_W

        def chunk(g, _):
            i0 = base + g * NB
            pltpu.sync_copy(idx_hbm.at[pl.ds(i0, NB)], idx_v)
            for b in range(NB):
                pltpu.async_copy(
                    table_hbm.at[idx_v.at[b]], rows_v.at[b], sem
                )
            for b in range(NB):
                pltpu.make_async_copy(
                    table_hbm.at[idx_v.at[b]], rows_v.at[b], sem
                ).wait()
            pltpu.sync_copy(rows_v, out_hbm.at[pl.ds(i0, NB)])
            return 0

        lax.fori_loop(0, N_CHUNKS, chunk, 0)

    return gather_k


def kernel(x, weight_fp):
    w128 = weight_fp.T.reshape(ROWS128, 128)

    maxabs = pl.pallas_call(
        _maxabs_body,
        grid=(N_BLOCKS,),
        in_specs=[pl.BlockSpec((BLK, 128), lambda i: (i, 0))],
        out_specs=pl.BlockSpec((1, 1), lambda i: (0, 0)),
        out_shape=jax.ShapeDtypeStruct((1, 1), jnp.float32),
    )(w128)

    w_q = pl.pallas_call(
        _quant_body,
        grid=(N_BLOCKS,),
        in_specs=[
            pl.BlockSpec((1, 1), lambda i: (0, 0)),
            pl.BlockSpec((BLK, 128), lambda i: (i, 0)),
        ],
        out_specs=pl.BlockSpec((BLK, 128), lambda i: (i, 0)),
        out_shape=jax.ShapeDtypeStruct((ROWS128, 128), jnp.float32),
    )(maxabs, w128)

    w_q = w_q.reshape(DIM, NUM_EMB).T
    return _make_gather()(w_q, x.astype(jnp.int32))


# speedup vs baseline: 6.2826x; 6.2826x over previous
import functools

import jax
import jax.numpy as jnp
from jax import lax
from jax.experimental import pallas as pl
from jax.experimental.pallas import tpu as pltpu
from jax.experimental.pallas import tpu_sc as plsc

NUM_EMB = 1000000
DIM = 32

BATCH = 16384
HIST = 50
NW = 32
B_PER_W = BATCH // NW
NB = 16
N_CHUNKS = B_PER_W // NB

R_PER_W = NUM_EMB // NW
SW_CH = 625
N_SW = R_PER_W // SW_CH
NBUF = 2

_MAGIC = 12582912.0


@functools.cache
def _sc_mesh():
    return plsc.VectorSubcoreMesh(core_axis_name="c", subcore_axis_name="s")


@functools.cache
def _make_sweep():
    @functools.partial(
        pl.kernel,
        mesh=_sc_mesh(),
        compiler_params=pltpu.CompilerParams(use_tc_tiling_on_sc=False, needs_layout_passes=False),
        out_type=jax.ShapeDtypeStruct((NW, 16), jnp.float32),
        scratch_types=[
            pltpu.VMEM((SW_CH, DIM), jnp.float32),
            pltpu.VMEM((SW_CH, DIM), jnp.float32),
            pltpu.VMEM((16,), jnp.float32),
            pltpu.SemaphoreType.DMA,
            pltpu.SemaphoreType.DMA,
        ],
    )
    def sweep_k(table_hbm, out_hbm, b0, b1, acc_v, sem0, sem1):
        wid = lax.axis_index("s") * 2 + lax.axis_index("c")
        r0 = wid * R_PER_W
        bufs = (b0, b1)
        sems = (sem0, sem1)

        for b in range(NBUF):
            pltpu.async_copy(
                table_hbm.at[pl.ds(r0 + b * SW_CH, SW_CH)], bufs[b], sems[b]
            )

        def outer(o, m):
            for b in range(NBUF):
                g = o * NBUF + b
                buf, sem = bufs[b], sems[b]
                pltpu.make_async_copy(
                    table_hbm.at[pl.ds(r0, SW_CH)], buf, sem
                ).wait()

                def row(r, mm):
                    lo = jnp.abs(buf[r, pl.ds(0, 16)])
                    hi = jnp.abs(buf[r, pl.ds(16, 16)])
                    return jnp.maximum(mm, jnp.maximum(lo, hi))

                m = lax.fori_loop(0, SW_CH, row, m)

                @pl.when(g + NBUF < N_SW)
                def _():
                    pltpu.async_copy(
                        table_hbm.at[pl.ds(r0 + (g + NBUF) * SW_CH, SW_CH)],
                        buf,
                        sem,
                    )
            return m

        m = lax.fori_loop(
            0, N_SW // NBUF, outer, jnp.zeros((16,), jnp.float32)
        )
        acc_v[...] = m
        pltpu.sync_copy(acc_v, out_hbm.at[wid])

    return sweep_k


@functools.cache
def _make_gather():
    @functools.partial(
        pl.kernel,
        mesh=_sc_mesh(),
        compiler_params=pltpu.CompilerParams(use_tc_tiling_on_sc=False, needs_layout_passes=False),
        out_type=jax.ShapeDtypeStruct((BATCH, HIST, DIM), jnp.float32),
        scratch_types=[
            pltpu.VMEM((NW, 16), jnp.float32),
            pltpu.VMEM((NB, HIST), jnp.int32),
            pltpu.VMEM((NB, HIST, DIM), jnp.float32),
            pltpu.SemaphoreType.DMA,
        ],
    )
    def gather_k(table_hbm, maxes_hbm, idx_hbm, out_hbm, mx_v, idx_v, rows_v, sem):
        wid = lax.axis_index("s") * 2 + lax.axis_index("c")
        base = wid * B_PER_W

        pltpu.sync_copy(maxes_hbm, mx_v)

        def red(i, m):
            return jnp.maximum(m, mx_v[i, pl.ds(0, 16)])

        m = lax.fori_loop(0, NW, red, jnp.zeros((16,), jnp.float32))
        scale = jnp.maximum(jnp.full((16,), jnp.max(m)) / 7.0, 1e-08)

        def chunk(g, _):
            i0 = base + g * NB
            pltpu.sync_copy(idx_hbm.at[pl.ds(i0, NB)], idx_v)
            for b in range(NB):
                pltpu.async_copy(
                    table_hbm.at[idx_v.at[b]], rows_v.at[b], sem
                )
            for b in range(NB):
                pltpu.make_async_copy(
                    table_hbm.at[idx_v.at[b]], rows_v.at[b], sem
                ).wait()

            def qrow(r, _c):
                i = r // HIST
                j = r % HIST
                for h in range(2):
                    v = rows_v[i, j, pl.ds(16 * h, 16)]
                    q = (v / scale + _MAGIC) - _MAGIC
                    q = jnp.clip(q, -8.0, 7.0) * scale
                    rows_v[i, j, pl.ds(16 * h, 16)] = q
                return 0

            lax.fori_loop(0, NB * HIST, qrow, 0)
            pltpu.sync_copy(rows_v, out_hbm.at[pl.ds(i0, NB)])
            return 0

        lax.fori_loop(0, N_CHUNKS, chunk, 0)

    return gather_k


def kernel(x, weight_fp):
    maxes = _make_sweep()(weight_fp)
    return _make_gather()(weight_fp, maxes, x.astype(jnp.int32))
